# baseline (speedup 1.0000x reference)
import functools

import jax
import jax.numpy as jnp
from jax.experimental import pallas as pl
from jax.experimental.pallas import tpu as pltpu
from jax.experimental.pallas import tpu_sc as plsc

STACK_DIM = 1024
STACK_SIZE = 16384

_POINTER = 0
_READ_ROW = (_POINTER + 1) % STACK_SIZE


@functools.partial(
    pl.kernel,
    mesh=plsc.ScalarSubcoreMesh(axis_name="c", num_cores=1),
    out_type=jax.ShapeDtypeStruct((STACK_DIM,), jnp.float32),
)
def _pop_row(x_hbm, stack_hbm, out_hbm):
    pltpu.sync_copy(stack_hbm.at[_READ_ROW], out_hbm)


def kernel(x, stack):
    return _pop_row(x, stack)


# speedup vs baseline: 1.0068x; 1.0068x over previous
import functools

import jax
import jax.numpy as jnp
from jax.experimental import pallas as pl
from jax.experimental.pallas import tpu as pltpu
from jax.experimental.pallas import tpu_sc as plsc

STACK_DIM = 1024
STACK_SIZE = 16384

_POINTER = 0
_READ_ROW = (_POINTER + 1) % STACK_SIZE


@functools.partial(
    pl.kernel,
    mesh=plsc.ScalarSubcoreMesh(axis_name="c", num_cores=1),
    out_type=jax.ShapeDtypeStruct((STACK_DIM,), jnp.float32),
)
def _pop_row(stack_hbm, out_hbm):
    pltpu.sync_copy(stack_hbm.at[_READ_ROW], out_hbm)


def kernel(x, stack):
    return _pop_row(stack)


# speedup vs baseline: 1.0151x; 1.0083x over previous
import functools

import jax
import jax.numpy as jnp
from jax.experimental import pallas as pl
from jax.experimental.pallas import tpu as pltpu
from jax.experimental.pallas import tpu_sc as plsc

STACK_DIM = 1024
STACK_SIZE = 16384

_POINTER = 0
_READ_ROW = (_POINTER + 1) % STACK_SIZE


@functools.partial(
    pl.kernel,
    mesh=plsc.ScalarSubcoreMesh(axis_name="c", num_cores=1),
    out_type=jax.ShapeDtypeStruct((STACK_DIM,), jnp.float32),
    compiler_params=pltpu.CompilerParams(skip_device_barrier=True),
)
def _pop_row(stack_hbm, out_hbm):
    pltpu.sync_copy(stack_hbm.at[_READ_ROW], out_hbm)


def kernel(x, stack):
    return _pop_row(stack)
